# baseline (speedup 1.0000x reference)
import functools

import jax
import jax.numpy as jnp
from jax import lax
from jax.experimental import pallas as pl
from jax.experimental.pallas import tpu as pltpu
from jax.experimental.pallas import tpu_sc as plsc

_TABLE_ROWS = 1024
_D_IN = 128
_D_HID = 512
_D_OUT = 512
_D_HALF = _D_OUT // 2
_BATCH = 16384


def _mlp_table_body(emb_ref, w1_ref, b1_ref, w2_ref, b2_ref, out_ref):
    h = jnp.dot(emb_ref[...], w1_ref[...], preferred_element_type=jnp.float32)
    h = h + b1_ref[...]
    h = h * lax.logistic(h)
    o = jnp.dot(h, w2_ref[...], preferred_element_type=jnp.float32)
    o = o + b2_ref[...]
    o = o * lax.logistic(o)
    lo = lax.convert_element_type(o[:, :_D_HALF], jnp.bfloat16)
    hi = lax.convert_element_type(o[:, _D_HALF:], jnp.bfloat16)
    lo_u = lax.bitcast_convert_type(lo, jnp.uint16).astype(jnp.uint32)
    hi_u = lax.bitcast_convert_type(hi, jnp.uint16).astype(jnp.uint32)
    out_ref[...] = lo_u | (hi_u << 16)


def _mlp_table(embedding, W1, b1, W2, b2):
    return pl.pallas_call(
        _mlp_table_body,
        out_shape=jax.ShapeDtypeStruct((_TABLE_ROWS, _D_HALF), jnp.uint32),
    )(embedding, W1, b1.reshape(1, _D_HID), W2, b2.reshape(1, _D_OUT))


_info = plsc.get_sparse_core_info()
_NC, _NS = _info.num_cores, _info.num_subcores
_NW = _NC * _NS
_BPW = _BATCH // _NW
_CHUNK = 64
_NCHUNK = _BPW // _CHUNK
_NBUF = 2
_LANES = 16
_KGRP = _D_HALF // _LANES

_sc_mesh = plsc.VectorSubcoreMesh(core_axis_name="c", subcore_axis_name="s")


@functools.partial(
    pl.kernel,
    mesh=_sc_mesh,
    out_type=jax.ShapeDtypeStruct((_BATCH, _D_OUT), jnp.float32),
    scratch_types=[
        pltpu.VMEM((_BPW,), jnp.int32),
        pltpu.VMEM((_NBUF, _CHUNK, _D_HALF), jnp.uint32),
        pltpu.VMEM((_NBUF, _CHUNK, _D_OUT), jnp.float32),
        pltpu.SemaphoreType.DMA((_NBUF,)),
        pltpu.SemaphoreType.DMA((_NBUF,)),
    ],
    compiler_params=pltpu.CompilerParams(needs_layout_passes=False),
)
def _sc_gather(table_hbm, idx_hbm, out_hbm, idx_v, packed_v, rows_v, gsem, osem):
    wid = lax.axis_index("s") * _NC + lax.axis_index("c")
    base = wid * _BPW
    pltpu.sync_copy(idx_hbm.at[pl.ds(base, _BPW)], idx_v)

    def fire_gather(c):
        return pltpu.async_copy(
            table_hbm.at[idx_v.at[pl.ds(c * _CHUNK, _CHUNK)]],
            packed_v.at[c % _NBUF],
            gsem.at[c % _NBUF],
        )

    def fire_out(c):
        return pltpu.async_copy(
            rows_v.at[c % _NBUF],
            out_hbm.at[pl.ds(base + c * _CHUNK, _CHUNK)],
            osem.at[c % _NBUF],
        )

    def unpack_chunk(b):
        src = packed_v.at[b]
        dst = rows_v.at[b]
        mask = jnp.uint32(0xFFFF0000)

        def row_body(r, carry):
            for k in range(_KGRP):
                w = src[r, pl.ds(k * _LANES, _LANES)]
                f_lo = plsc.bitcast(w << 16, jnp.float32)
                f_hi = plsc.bitcast(w & mask, jnp.float32)
                dst[r, pl.ds(k * _LANES, _LANES)] = f_lo
                dst[r, pl.ds(_D_HALF + k * _LANES, _LANES)] = f_hi
            return carry

        lax.fori_loop(0, _CHUNK, row_body, 0)

    gathers = {0: fire_gather(0)}
    outs = {}
    for c in range(_NCHUNK):
        nxt = c + 1
        if nxt < _NCHUNK:
            gathers[nxt] = fire_gather(nxt)
        gathers[c].wait()
        if c - _NBUF in outs:
            outs[c - _NBUF].wait()
        unpack_chunk(c % _NBUF)
        outs[c] = fire_out(c)
    for c in range(_NCHUNK - _NBUF, _NCHUNK):
        outs[c].wait()


def kernel(diffusion_step, embedding, W1, b1, W2, b2):
    emb_pad = jnp.pad(embedding, ((0, _TABLE_ROWS - embedding.shape[0]), (0, 0)))
    table = _mlp_table(emb_pad, W1, b1, W2, b2)
    idx = diffusion_step.astype(jnp.int32)
    return _sc_gather(table, idx)


# speedup vs baseline: 1.3543x; 1.3543x over previous
import functools

import jax
import jax.numpy as jnp
from jax import lax
from jax.experimental import pallas as pl
from jax.experimental.pallas import tpu as pltpu
from jax.experimental.pallas import tpu_sc as plsc

_TABLE_ROWS = 1024
_LIVE_ROWS = 1000
_D_IN = 128
_D_HID = 512
_D_OUT = 512
_BATCH = 16384


def _mlp_table_body(emb_ref, w1_ref, b1_ref, w2_ref, b2_ref, out_ref):
    h = jnp.dot(emb_ref[...], w1_ref[...], preferred_element_type=jnp.float32)
    h = h + b1_ref[...]
    h = h * lax.logistic(h)
    o = jnp.dot(h, w2_ref[...], preferred_element_type=jnp.float32)
    o = o + b2_ref[...]
    out_ref[pl.ds(0, _LIVE_ROWS), :] = o * lax.logistic(o)


def _mlp_table(embedding, W1, b1, W2, b2):
    return pl.pallas_call(
        _mlp_table_body,
        out_shape=jax.ShapeDtypeStruct((_TABLE_ROWS, _D_OUT), jnp.float32),
    )(embedding, W1, b1.reshape(1, _D_HID), W2, b2.reshape(1, _D_OUT))


_info = plsc.get_sparse_core_info()
_NC, _NS = _info.num_cores, _info.num_subcores
_NW = _NC * _NS
_BPW = _BATCH // _NW
_CHUNK = 64
_NCHUNK = _BPW // _CHUNK
_NBUF = 3

_sc_mesh = plsc.VectorSubcoreMesh(core_axis_name="c", subcore_axis_name="s")


@functools.partial(
    pl.kernel,
    mesh=_sc_mesh,
    out_type=jax.ShapeDtypeStruct((_BATCH, _D_OUT), jnp.float32),
    scratch_types=[
        pltpu.VMEM((_BPW,), jnp.int32),
        pltpu.VMEM((_NBUF, _CHUNK, _D_OUT), jnp.float32),
        pltpu.SemaphoreType.DMA((_NBUF,)),
        pltpu.SemaphoreType.DMA((_NBUF,)),
    ],
)
def _sc_gather(table_hbm, idx_hbm, out_hbm, idx_v, rows_v, gsem, osem):
    wid = lax.axis_index("s") * _NC + lax.axis_index("c")
    base = wid * _BPW
    pltpu.sync_copy(idx_hbm.at[pl.ds(base, _BPW)], idx_v)

    def fire_gather(c):
        return pltpu.async_copy(
            table_hbm.at[idx_v.at[pl.ds(c * _CHUNK, _CHUNK)]],
            rows_v.at[c % _NBUF],
            gsem.at[c % _NBUF],
        )

    def fire_out(c):
        return pltpu.async_copy(
            rows_v.at[c % _NBUF],
            out_hbm.at[pl.ds(base + c * _CHUNK, _CHUNK)],
            osem.at[c % _NBUF],
        )

    gathers = {0: fire_gather(0)}
    outs = {}
    for c in range(_NCHUNK):
        nxt = c + 1
        if nxt < _NCHUNK:
            if nxt - _NBUF in outs:
                outs[nxt - _NBUF].wait()
            gathers[nxt] = fire_gather(nxt)
        gathers[c].wait()
        outs[c] = fire_out(c)
    for c in range(_NCHUNK - _NBUF, _NCHUNK):
        outs[c].wait()


def kernel(diffusion_step, embedding, W1, b1, W2, b2):
    table = _mlp_table(embedding, W1, b1, W2, b2)
    idx = diffusion_step.astype(jnp.int32)
    return _sc_gather(table, idx)


# speedup vs baseline: 1.3545x; 1.0002x over previous
import functools

import jax
import jax.numpy as jnp
from jax import lax
from jax.experimental import pallas as pl
from jax.experimental.pallas import tpu as pltpu
from jax.experimental.pallas import tpu_sc as plsc

_TABLE_ROWS = 1024
_LIVE_ROWS = 1000
_D_IN = 128
_D_HID = 512
_D_OUT = 512
_BATCH = 16384


def _mlp_table_body(emb_ref, w1_ref, b1_ref, w2_ref, b2_ref, out_ref):
    h = jnp.dot(emb_ref[...], w1_ref[...], preferred_element_type=jnp.float32)
    h = h + b1_ref[...]
    h = h * lax.logistic(h)
    o = jnp.dot(h, w2_ref[...], preferred_element_type=jnp.float32)
    o = o + b2_ref[...]
    out_ref[pl.ds(0, _LIVE_ROWS), :] = o * lax.logistic(o)


def _mlp_table(embedding, W1, b1, W2, b2):
    return pl.pallas_call(
        _mlp_table_body,
        out_shape=jax.ShapeDtypeStruct((_TABLE_ROWS, _D_OUT), jnp.float32),
    )(embedding, W1, b1.reshape(1, _D_HID), W2, b2.reshape(1, _D_OUT))


_info = plsc.get_sparse_core_info()
_NC, _NS = _info.num_cores, _info.num_subcores
_NW = _NC * _NS
_BPW = _BATCH // _NW
_CHUNK = 32
_NCHUNK = _BPW // _CHUNK
_NBUF = 2
_NSLOT = 2

_sc_mesh = plsc.VectorSubcoreMesh(core_axis_name="c", subcore_axis_name="s")


@functools.partial(
    pl.kernel,
    mesh=_sc_mesh,
    out_type=jax.ShapeDtypeStruct((_BATCH, _D_OUT), jnp.float32),
    scratch_types=[
        pltpu.VMEM((_BPW,), jnp.int32),
        pltpu.VMEM((_NBUF, _CHUNK, _D_OUT), jnp.float32),
        pltpu.VMEM_SHARED((_NS, _NSLOT, _CHUNK, _D_OUT), jnp.float32),
        pltpu.SemaphoreType.DMA((_NBUF,)),
        pltpu.SemaphoreType.DMA((_NSLOT,)),
    ],
)
def _sc_gather(table_hbm, idx_hbm, out_hbm, idx_v, rows_v, slots_sh, gsem, ssem):
    sid = lax.axis_index("s")
    wid = sid * _NC + lax.axis_index("c")
    base = wid * _BPW
    pltpu.sync_copy(idx_hbm.at[pl.ds(base, _BPW)], idx_v)

    def fire_gather(c):
        return pltpu.async_copy(
            table_hbm.at[idx_v.at[pl.ds(c * _CHUNK, _CHUNK)]],
            rows_v.at[c % _NBUF],
            gsem.at[c % _NBUF],
        )

    def fire_out(c):
        return pltpu.async_copy(
            slots_sh.at[sid, c % _NSLOT],
            out_hbm.at[pl.ds(base + c * _CHUNK, _CHUNK)],
            ssem.at[c % _NSLOT],
        )

    gathers = {0: fire_gather(0)}
    outs = {}
    for c in range(_NCHUNK):
        nxt = c + 1
        if nxt < _NCHUNK:
            gathers[nxt] = fire_gather(nxt)
        gathers[c].wait()
        if c - _NSLOT in outs:
            outs[c - _NSLOT].wait()
        pltpu.sync_copy(rows_v.at[c % _NBUF], slots_sh.at[sid, c % _NSLOT])
        outs[c] = fire_out(c)
    for c in range(_NCHUNK - _NSLOT, _NCHUNK):
        outs[c].wait()


def kernel(diffusion_step, embedding, W1, b1, W2, b2):
    table = _mlp_table(embedding, W1, b1, W2, b2)
    idx = diffusion_step.astype(jnp.int32)
    return _sc_gather(table, idx)


# speedup vs baseline: 1.3610x; 1.0047x over previous
import functools

import jax
import jax.numpy as jnp
from jax import lax
from jax.experimental import pallas as pl
from jax.experimental.pallas import tpu as pltpu
from jax.experimental.pallas import tpu_sc as plsc

_TABLE_ROWS = 1024
_LIVE_ROWS = 1000
_D_IN = 128
_D_HID = 512
_D_OUT = 512
_BATCH = 16384


def _mlp_table_body(emb_ref, w1_ref, b1_ref, w2_ref, b2_ref, out_ref):
    h = jnp.dot(emb_ref[...], w1_ref[...], preferred_element_type=jnp.float32)
    h = h + b1_ref[...]
    h = h * lax.logistic(h)
    o = jnp.dot(h, w2_ref[...], preferred_element_type=jnp.float32)
    o = o + b2_ref[...]
    out_ref[pl.ds(0, _LIVE_ROWS), :] = o * lax.logistic(o)


def _mlp_table(embedding, W1, b1, W2, b2):
    return pl.pallas_call(
        _mlp_table_body,
        out_shape=jax.ShapeDtypeStruct((_TABLE_ROWS, _D_OUT), jnp.float32),
    )(embedding, W1, b1.reshape(1, _D_HID), W2, b2.reshape(1, _D_OUT))


_info = plsc.get_sparse_core_info()
_NC, _NS = _info.num_cores, _info.num_subcores
_NW = _NC * _NS
_BPW = _BATCH // _NW
_CHUNK = 32
_NCHUNK = _BPW // _CHUNK
_NBUF = 3
_NSLOT = 4

_sc_mesh = plsc.VectorSubcoreMesh(core_axis_name="c", subcore_axis_name="s")


@functools.partial(
    pl.kernel,
    mesh=_sc_mesh,
    out_type=jax.ShapeDtypeStruct((_BATCH, _D_OUT), jnp.float32),
    scratch_types=[
        pltpu.VMEM((_BPW,), jnp.int32),
        pltpu.VMEM((_NBUF, _CHUNK, _D_OUT), jnp.float32),
        pltpu.VMEM_SHARED((_NS, _NSLOT, _CHUNK, _D_OUT), jnp.float32),
        pltpu.SemaphoreType.DMA((_NBUF,)),
        pltpu.SemaphoreType.DMA((_NSLOT,)),
        pltpu.SemaphoreType.DMA((_NSLOT,)),
    ],
)
def _sc_gather(table_hbm, idx_hbm, out_hbm, idx_v, rows_v, slots_sh, gsem, xsem, ssem):
    sid = lax.axis_index("s")
    wid = sid * _NC + lax.axis_index("c")
    base = wid * _BPW
    pltpu.sync_copy(idx_hbm.at[pl.ds(base, _BPW)], idx_v)

    def fire_gather(c):
        return pltpu.async_copy(
            table_hbm.at[idx_v.at[pl.ds(c * _CHUNK, _CHUNK)]],
            rows_v.at[c % _NBUF],
            gsem.at[c % _NBUF],
        )

    def fire_cross(c):
        return pltpu.async_copy(
            rows_v.at[c % _NBUF],
            slots_sh.at[sid, c % _NSLOT],
            xsem.at[c % _NSLOT],
        )

    def fire_out(c):
        return pltpu.async_copy(
            slots_sh.at[sid, c % _NSLOT],
            out_hbm.at[pl.ds(base + c * _CHUNK, _CHUNK)],
            ssem.at[c % _NSLOT],
        )

    gathers = {0: fire_gather(0)}
    crosses = {}
    outs = {}
    for c in range(_NCHUNK):
        if c - 1 in crosses:
            crosses[c - 1].wait()
            outs[c - 1] = fire_out(c - 1)
        nxt = c + 1
        if nxt < _NCHUNK:
            gathers[nxt] = fire_gather(nxt)
        gathers[c].wait()
        if c - _NSLOT in outs:
            outs[c - _NSLOT].wait()
        crosses[c] = fire_cross(c)
    last = _NCHUNK - 1
    crosses[last].wait()
    outs[last] = fire_out(last)
    for c in range(_NCHUNK - _NSLOT, _NCHUNK):
        outs[c].wait()


def kernel(diffusion_step, embedding, W1, b1, W2, b2):
    table = _mlp_table(embedding, W1, b1, W2, b2)
    idx = diffusion_step.astype(jnp.int32)
    return _sc_gather(table, idx)
